# Initial kernel scaffold; baseline (speedup 1.0000x reference)
import jax
import jax.numpy as jnp
from jax.experimental import pallas as pl


def kernel(edge_index, in_feat, W, alphas):
    raise NotImplementedError("write your pallas kernel here")


# speedup vs baseline: 21.5297x; 21.5297x over previous
import functools
import jax
import jax.numpy as jnp
from jax import lax
from jax.experimental import pallas as pl
from jax.experimental.pallas import tpu as pltpu
from jax.experimental.pallas import tpu_sc as plsc

NN = 10000
EE = 320000
DD = 128
NC = 2
NS = 16
NPAD = 10240
CHUNK = 64
CPT = 160
EPAD = NC * NS * CPT * CHUNK
RPT = NPAD // NS

_mesh = plsc.VectorSubcoreMesh(
    core_axis_name="c", subcore_axis_name="s", num_cores=NC, num_subcores=NS)


@functools.partial(
    pl.kernel,
    out_type=jax.ShapeDtypeStruct((NC, NPAD, DD), jnp.float32),
    mesh=_mesh,
    scratch_types=[
        pltpu.VMEM((CPT, CHUNK), jnp.int32),
        pltpu.VMEM((CHUNK, DD), jnp.float32),
        pltpu.VMEM_SHARED((NPAD, DD), jnp.float32),
        pltpu.SemaphoreType.DMA,
    ],
)
def _deg_kernel(dst_hbm, zeros_hbm, out_hbm, dstb, ones, acc, sem):
    c = lax.axis_index("c")
    s = lax.axis_index("s")
    wid = c * NS + s

    def _fill(i, _):
        def _col(j, _):
            ones[i, pl.ds(j * 16, 16)] = jnp.ones((16,), jnp.float32)
            return 0
        lax.fori_loop(0, DD // 16, _col, 0)
        return 0
    lax.fori_loop(0, CHUNK, _fill, 0)

    pltpu.sync_copy(zeros_hbm.at[pl.ds(s * RPT, RPT)],
                    acc.at[pl.ds(s * RPT, RPT)])
    plsc.subcore_barrier()

    pltpu.sync_copy(dst_hbm.at[pl.ds(wid * CPT, CPT)], dstb)

    def _edges(i, _):
        for k in range(4):
            pltpu.async_copy(ones, acc.at[dstb.at[i * 4 + k]], sem, add=True)
        for k in range(4):
            pltpu.make_async_copy(ones, acc.at[dstb.at[0]], sem).wait()
        return 0
    lax.fori_loop(0, CPT // 4, _edges, 0)

    plsc.subcore_barrier()
    pltpu.sync_copy(acc.at[pl.ds(s * RPT, RPT)],
                    out_hbm.at[c, pl.ds(s * RPT, RPT)])


_NBUF = 4
_PASSES = 4
_PCH = CPT // _PASSES


@functools.partial(
    pl.kernel,
    out_type=jax.ShapeDtypeStruct((NC, NPAD, DD), jnp.float32),
    mesh=_mesh,
    scratch_types=[
        pltpu.VMEM((_PCH, CHUNK), jnp.int32),
        pltpu.VMEM((_PCH, CHUNK), jnp.int32),
        pltpu.VMEM_SHARED((NPAD, DD), jnp.float32),
    ]
    + [pltpu.VMEM((CHUNK, DD), jnp.float32) for _ in range(_NBUF)]
    + [pltpu.SemaphoreType.DMA for _ in range(2 * _NBUF)],
)
def _prop_kernel(y_hbm, src_hbm, dst_hbm, zeros_hbm, out_hbm,
                 srcb, dstb, acc, *bufs_and_sems):
    rows = bufs_and_sems[:_NBUF]
    gsem = bufs_and_sems[_NBUF:2 * _NBUF]
    ssem = bufs_and_sems[2 * _NBUF:]
    c = lax.axis_index("c")
    s = lax.axis_index("s")
    wid = c * NS + s

    pltpu.sync_copy(zeros_hbm.at[pl.ds(s * RPT, RPT)],
                    acc.at[pl.ds(s * RPT, RPT)])
    plsc.subcore_barrier()

    def gstart(j, k):
        pltpu.async_copy(y_hbm.at[srcb.at[j]], rows[k], gsem[k])

    def gwait(k):
        pltpu.make_async_copy(y_hbm.at[srcb.at[0]], rows[k], gsem[k]).wait()

    def sstart(j, k):
        pltpu.async_copy(rows[k], acc.at[dstb.at[j]], ssem[k], add=True)

    def swait(k):
        pltpu.make_async_copy(rows[k], acc.at[dstb.at[0]], ssem[k]).wait()

    for p in range(_PASSES):
        pltpu.sync_copy(src_hbm.at[pl.ds(wid * CPT + p * _PCH, _PCH)], srcb)
        pltpu.sync_copy(dst_hbm.at[pl.ds(wid * CPT + p * _PCH, _PCH)], dstb)

        for k in range(_NBUF):
            gstart(k, k)

        ngrp = _PCH // _NBUF

        def _grp(i, _):
            base = i * _NBUF
            for k in range(_NBUF):
                gwait(k)
                sstart(base + k, k)
            for k in range(_NBUF):
                swait(k)
                gstart(base + _NBUF + k, k)
            return 0
        lax.fori_loop(0, ngrp - 1, _grp, 0)

        base = (ngrp - 1) * _NBUF
        for k in range(_NBUF):
            gwait(k)
            sstart(base + k, k)
        for k in range(_NBUF):
            swait(k)

    plsc.subcore_barrier()
    pltpu.sync_copy(acc.at[pl.ds(s * RPT, RPT)],
                    out_hbm.at[c, pl.ds(s * RPT, RPT)])


_BR = 1280


def _tcmm_body(x_ref, w_ref, y_ref):
    y_ref[...] = jnp.dot(x_ref[...], w_ref[...],
                         preferred_element_type=jnp.float32)


def _tcmm(x, w):
    return pl.pallas_call(
        _tcmm_body,
        grid=(NPAD // _BR,),
        in_specs=[
            pl.BlockSpec((_BR, DD), lambda i: (i, 0)),
            pl.BlockSpec((DD, DD), lambda i: (0, 0)),
        ],
        out_specs=pl.BlockSpec((_BR, DD), lambda i: (i, 0)),
        out_shape=jax.ShapeDtypeStruct((NPAD, DD), jnp.float32),
    )(x, w)


def _tcscale_body(x_ref, dv_ref, y_ref):
    y_ref[...] = x_ref[...] * dv_ref[...]


def _tcscale(x, dvb):
    return pl.pallas_call(
        _tcscale_body,
        grid=(NPAD // _BR,),
        in_specs=[
            pl.BlockSpec((_BR, DD), lambda i: (i, 0)),
            pl.BlockSpec((_BR, DD), lambda i: (i, 0)),
        ],
        out_specs=pl.BlockSpec((_BR, DD), lambda i: (i, 0)),
        out_shape=jax.ShapeDtypeStruct((NPAD, DD), jnp.float32),
    )(x, dvb)


def _tc2_body(p_ref, dv_ref, h1_ref, y1_ref):
    dv = dv_ref[...]
    h = (p_ref[0] + p_ref[1]) * dv
    h1_ref[...] = h
    y1_ref[...] = h * dv


def _tc2(p, dvb):
    return pl.pallas_call(
        _tc2_body,
        grid=(NPAD // _BR,),
        in_specs=[
            pl.BlockSpec((NC, _BR, DD), lambda i: (0, i, 0)),
            pl.BlockSpec((_BR, DD), lambda i: (i, 0)),
        ],
        out_specs=[
            pl.BlockSpec((_BR, DD), lambda i: (i, 0)),
            pl.BlockSpec((_BR, DD), lambda i: (i, 0)),
        ],
        out_shape=[
            jax.ShapeDtypeStruct((NPAD, DD), jnp.float32),
            jax.ShapeDtypeStruct((NPAD, DD), jnp.float32),
        ],
    )(p, dvb)


_BR3 = 1000


def _tc3_body(q_ref, h1_ref, dv_ref, al_ref, out_ref):
    h2 = (q_ref[0] + q_ref[1]) * dv_ref[...]
    out_ref[...] = al_ref[0] * h1_ref[...] + al_ref[1] * h2


def _tc3(q, h1, dvb, alphas):
    return pl.pallas_call(
        _tc3_body,
        grid=(NN // _BR3,),
        in_specs=[
            pl.BlockSpec((NC, _BR3, DD), lambda i: (0, i, 0)),
            pl.BlockSpec((_BR3, DD), lambda i: (i, 0)),
            pl.BlockSpec((_BR3, DD), lambda i: (i, 0)),
            pl.BlockSpec(memory_space=pltpu.SMEM),
        ],
        out_specs=pl.BlockSpec((_BR3, DD), lambda i: (i, 0)),
        out_shape=jax.ShapeDtypeStruct((NN, DD), jnp.float32),
    )(q, h1, dvb, alphas)


def kernel(edge_index, in_feat, W, alphas):
    src = edge_index[0]
    dst = edge_index[1]
    npad_e = EPAD - EE
    padidx = NN + (jnp.arange(npad_e, dtype=jnp.int32) % 16)
    srcp = jnp.concatenate([src, padidx]).reshape(-1, CHUNK)
    dstp = jnp.concatenate([dst, padidx]).reshape(-1, CHUNK)

    zerosd = jnp.zeros((NPAD, DD), jnp.float32)

    degp = _deg_kernel(dstp, zerosd)
    deg = jnp.maximum(degp[0, :, 0] + degp[1, :, 0], 1.0)
    dv = lax.rsqrt(deg)
    dvb = jnp.broadcast_to(dv[:, None], (NPAD, DD))

    in_pad = jnp.zeros((NPAD, DD), jnp.float32).at[:NN].set(in_feat)
    x0 = _tcmm(in_pad, W)
    y0 = _tcscale(x0, dvb)
    p = _prop_kernel(y0, srcp, dstp, zerosd)
    h1, y1 = _tc2(p, dvb)
    q = _prop_kernel(y1, srcp, dstp, zerosd)
    return _tc3(q, h1, dvb, alphas)
